# baseline (speedup 1.0000x reference)
import functools

import jax
import jax.numpy as jnp
from jax import lax
from jax.experimental import pallas as pl
from jax.experimental.pallas import tpu as pltpu
from jax.experimental.pallas import tpu_sc as plsc

F32 = jnp.float32
NC, NS = 2, 16
CE = 40
NODE_SUB = 25


def _node_proj_body(x, wsg, bsg, wdg, bdg, wdu, bdu, wsu, bsu,
                    osg, odg, odu, osu):
    xv = x[...]

    def proj(w, b):
        return lax.dot_general(xv, w[...], (((1,), (1,)), ((), ())),
                               preferred_element_type=F32) + b[...]

    osg[...] = proj(wsg, bsg)
    odg[...] = proj(wdg, bdg)
    odu[...] = proj(wdu, bdu)
    osu[...] = proj(wsu, bsu)


def _edge_proj_body(x, w, b, o):
    xv = x[...]
    peg = lax.dot_general(xv, w[...], (((1,), (1,)), ((), ())),
                          preferred_element_type=F32) + b[...]
    h = xv.shape[1] // 2
    o[0] = jnp.concatenate([peg[:, :h], xv[:, :h]], axis=1)
    o[1] = jnp.concatenate([peg[:, h:], xv[:, h:]], axis=1)


def _sc_body(N, E, ti0, ti1, tj0, tj1, psut, node_t, pe2, ij_hbm,
             yout, xout,
             acc2,
             ijb0, ijb1, ijb2, ijb3,
             g_sg0, g_sg1, g_dj0, g_dj1,
             sm0, sm1, pe0, pe1, yo0, yo1,
             ep_sm, ep_psu, ep_node,
             sem_ij, sem_in0, sem_in1, sem_out0, sem_out1, sem_sc0, sem_sc1):
    c = lax.axis_index("c")
    s = lax.axis_index("s")
    cN = c * N
    c64 = c * 64
    zero16 = jnp.zeros((16,), F32)

    ijb = (ijb0, ijb1, ijb2, ijb3)
    g_sg = (g_sg0, g_sg1)
    g_dj = (g_dj0, g_dj1)
    sm = (sm0, sm1)
    pe = (pe0, pe1)
    yo = (yo0, yo1)
    sem_in = (sem_in0, sem_in1)
    sem_out = (sem_out0, sem_out1)
    sem_sc = (sem_sc0, sem_sc1)

    npt = N // NS
    ept = E // NS
    nchunks = ept // CE
    nsub = npt // NODE_SUB
    ebase0 = s * ept

    def zrow(r, x_):
        for q in range(8):
            ep_sm[r, pl.ds(q * 16, 16)] = zero16
        return x_
    lax.fori_loop(0, NODE_SUB, zrow, None)
    for q in range(nsub):
        pltpu.sync_copy(ep_sm, acc2.at[pl.ds(s * npt + q * NODE_SUB, NODE_SUB)])
    plsc.subcore_barrier()

    nct = ept // CE
    kbase0 = s * nct

    def issue_ij(k, q4):
        pltpu.async_copy(ij_hbm.at[kbase0 + k], ijb[q4], sem_ij)

    def wait_ij(q4):
        pltpu.make_async_copy(ij_hbm.at[kbase0], ijb[q4], sem_ij).wait()

    def issue_in(k, p, q4):
        base = ebase0 + k * CE

        @pl.when(c == 0)
        def _():
            pltpu.async_copy(ti0.at[ijb[q4].at[0]], g_sg[p], sem_in[p])
            pltpu.async_copy(tj0.at[ijb[q4].at[1]], g_dj[p], sem_in[p])

        @pl.when(c == 1)
        def _():
            pltpu.async_copy(ti1.at[ijb[q4].at[0]], g_sg[p], sem_in[p])
            pltpu.async_copy(tj1.at[ijb[q4].at[1]], g_dj[p], sem_in[p])

        pltpu.async_copy(pe2.at[pl.ds(c * E + base, CE)], pe[p], sem_in[p])

    def wait_in(p):
        pltpu.make_async_copy(ti0.at[ijb[0].at[0]], g_sg[p], sem_in[p]).wait()
        pltpu.make_async_copy(tj0.at[ijb[0].at[1]], g_dj[p], sem_in[p]).wait()
        pltpu.make_async_copy(pe2.at[pl.ds(0, CE)], pe[p], sem_in[p]).wait()

    def compute(p):
        def row2(t, x_):
            vals = []
            for rr in range(2):
                r = 2 * t + rr
                ys = [g_sg[p][r, pl.ds(q * 16, 16)]
                      + g_dj[p][r, pl.ds(q * 16, 16)]
                      + pe[p][r, pl.ds(q * 16, 16)] for q in range(4)]
                es = [pe[p][r, pl.ds(64 + q * 16, 16)] for q in range(4)]
                dus = [g_dj[p][r, pl.ds(64 + q * 16, 16)] for q in range(4)]
                vals.append((r, ys, es, dus))
            for r, ys, es, dus in vals:
                for q in range(4):
                    sg = 1.0 / (1.0 + jnp.exp(-ys[q]))
                    sm[p][r, pl.ds(q * 16, 16)] = sg
                    sm[p][r, pl.ds(64 + q * 16, 16)] = dus[q] * sg
                    yo[p][r, pl.ds(q * 16, 16)] = es[q] + ys[q] * sg
            return x_
        lax.fori_loop(0, CE // 2, row2, None)

    def issue_out(k, p, q4):
        base = ebase0 + k * CE
        pltpu.async_copy(yo[p], yout.at[pl.ds(base, CE), pl.ds(c64, 64)], sem_out[p])
        pltpu.async_copy(sm[p], acc2.at[ijb[q4].at[0]], sem_sc[p], add=True)

    def wait_out(p):
        pltpu.make_async_copy(yo[p], yout.at[pl.ds(0, CE), pl.ds(c64, 64)], sem_out[p]).wait()

    def wait_sc(p):
        pltpu.make_async_copy(sm[p], acc2.at[ijb[0].at[0]], sem_sc[p]).wait()

    issue_ij(0, 0)
    issue_ij(1, 1)
    wait_ij(0)
    issue_in(0, 0, 0)

    wait_ij(1)
    issue_in(1, 1, 1)
    wait_in(0)
    compute(0)
    issue_out(0, 0, 0)
    issue_ij(2, 2)

    wait_ij(2)
    wait_out(0)
    issue_in(2, 0, 2)
    wait_in(1)
    compute(1)
    issue_out(1, 1, 1)
    issue_ij(3, 3)

    def superstep(t, x_):
        k0 = 2 + t * 4

        def stage(koff, p, q4):
            k = k0 + koff
            wait_ij((q4 + 1) % 4)
            wait_out(1 - p)
            issue_in(k + 1, 1 - p, (q4 + 1) % 4)
            wait_in(p)
            wait_sc(p)
            compute(p)
            issue_out(k, p, q4)
            issue_ij(k + 2, (q4 + 2) % 4)

        stage(0, 0, 2)
        stage(1, 1, 3)
        stage(2, 0, 0)
        stage(3, 1, 1)
        return x_

    lax.fori_loop(0, (nchunks - 4) // 4, superstep, None)

    wait_ij(3)
    wait_out(1)
    issue_in(nchunks - 1, 1, 3)
    wait_in(0)
    wait_sc(0)
    compute(0)
    issue_out(nchunks - 2, 0, 2)

    wait_in(1)
    wait_sc(1)
    compute(1)
    issue_out(nchunks - 1, 1, 3)

    wait_out(0)
    wait_out(1)
    wait_sc(0)
    wait_sc(1)
    plsc.subcore_barrier()

    nbase0 = s * npt

    def ep(q, x_):
        nb = nbase0 + q * NODE_SUB
        pltpu.sync_copy(acc2.at[pl.ds(nb, NODE_SUB)], ep_sm)
        pltpu.sync_copy(psut.at[pl.ds(cN + nb, NODE_SUB)], ep_psu)
        pltpu.sync_copy(node_t.at[pl.ds(cN + nb, NODE_SUB)], ep_node)

        def nrow(r, y_):
            sig4 = [ep_sm[r, pl.ds(q4 * 16, 16)] for q4 in range(4)]
            m4 = [ep_sm[r, pl.ds(64 + q4 * 16, 16)] for q4 in range(4)]
            psu4 = [ep_psu[r, pl.ds(q4 * 16, 16)] for q4 in range(4)]
            nd4 = [ep_node[r, pl.ds(q4 * 16, 16)] for q4 in range(4)]
            for q4 in range(4):
                t = psu4[q4] + m4[q4] / (sig4[q4] + 1e-6)
                st = 1.0 / (1.0 + jnp.exp(-t))
                ep_node[r, pl.ds(q4 * 16, 16)] = nd4[q4] + t * st
            return y_
        lax.fori_loop(0, NODE_SUB, nrow, None)

        pltpu.sync_copy(ep_node, xout.at[pl.ds(cN + nb, NODE_SUB)])
        return x_

    lax.fori_loop(0, nsub, ep, None)


def kernel(node_feats, edge_feats, i, j, W_sg, b_sg, W_dg, b_dg,
           W_eg, b_eg, W_su, b_su, W_du, b_du):
    B, N, D = node_feats.shape
    E = edge_feats.shape[1]
    node2 = node_feats.reshape(N, D)
    edge2 = edge_feats.reshape(E, D)
    ij_hbm = jnp.stack([i.astype(jnp.int32).reshape(E // CE, CE),
                        j.astype(jnp.int32).reshape(E // CE, CE)], axis=1)
    h = D // 2

    BN = 400
    wspec = pl.BlockSpec((D, D), lambda nb: (0, 0))
    bspec = pl.BlockSpec((1, D), lambda nb: (0, 0))
    outs = pl.pallas_call(
        _node_proj_body,
        grid=(N // BN,),
        in_specs=[pl.BlockSpec((BN, D), lambda nb: (nb, 0)),
                  wspec, bspec, wspec, bspec, wspec, bspec, wspec, bspec],
        out_specs=[pl.BlockSpec((BN, D), lambda nb: (nb, 0))] * 4,
        out_shape=[jax.ShapeDtypeStruct((N, D), F32)] * 4,
    )(node2, W_sg, b_sg.reshape(1, D), W_dg, b_dg.reshape(1, D),
      W_du, b_du.reshape(1, D), W_su, b_su.reshape(1, D))
    p_sg, p_dg, p_du, p_su = outs

    ti0 = p_sg[:, :h]
    ti1 = p_sg[:, h:]
    tj0 = jnp.concatenate([p_dg[:, :h], p_du[:, :h]], axis=1)
    tj1 = jnp.concatenate([p_dg[:, h:], p_du[:, h:]], axis=1)
    psut = p_su.reshape(N, 2, h).transpose(1, 0, 2).reshape(2 * N, h)
    node_t = node2.reshape(N, 2, h).transpose(1, 0, 2).reshape(2 * N, h)

    BE = 1000
    pe = pl.pallas_call(
        _edge_proj_body,
        grid=(E // BE,),
        in_specs=[pl.BlockSpec((BE, D), lambda nb: (nb, 0)),
                  pl.BlockSpec((D, D), lambda nb: (0, 0)),
                  pl.BlockSpec((1, D), lambda nb: (0, 0))],
        out_specs=pl.BlockSpec((2, BE, D), lambda nb: (0, nb, 0)),
        out_shape=jax.ShapeDtypeStruct((2, E, D), F32),
    )(edge2, W_eg, b_eg.reshape(1, D))
    pe2 = pe.reshape(2 * E, D)

    mesh = plsc.VectorSubcoreMesh(core_axis_name="c", subcore_axis_name="s",
                                  num_cores=NC, num_subcores=NS)
    sc_fn = pl.kernel(
        functools.partial(_sc_body, N, E),
        out_type=[jax.ShapeDtypeStruct((E, D), F32),
                  jax.ShapeDtypeStruct((2 * N, h), F32)],
        mesh=mesh,
        compiler_params=pltpu.CompilerParams(use_tc_tiling_on_sc=False),
        scratch_types=[
            pltpu.VMEM_SHARED((N, D), F32),
            pltpu.VMEM((2, CE), jnp.int32),
            pltpu.VMEM((2, CE), jnp.int32),
            pltpu.VMEM((2, CE), jnp.int32),
            pltpu.VMEM((2, CE), jnp.int32),
            pltpu.VMEM((CE, h), F32),
            pltpu.VMEM((CE, h), F32),
            pltpu.VMEM((CE, D), F32),
            pltpu.VMEM((CE, D), F32),
            pltpu.VMEM((CE, D), F32),
            pltpu.VMEM((CE, D), F32),
            pltpu.VMEM((CE, D), F32),
            pltpu.VMEM((CE, D), F32),
            pltpu.VMEM((CE, h), F32),
            pltpu.VMEM((CE, h), F32),
            pltpu.VMEM((NODE_SUB, D), F32),
            pltpu.VMEM((NODE_SUB, h), F32),
            pltpu.VMEM((NODE_SUB, h), F32),
            pltpu.SemaphoreType.DMA,
            pltpu.SemaphoreType.DMA,
            pltpu.SemaphoreType.DMA,
            pltpu.SemaphoreType.DMA,
            pltpu.SemaphoreType.DMA,
            pltpu.SemaphoreType.DMA,
            pltpu.SemaphoreType.DMA,
        ],
    )
    yout, xout = sc_fn(ti0, ti1, tj0, tj1, psut, node_t, pe2, ij_hbm)

    x = xout.reshape(2, N, h).transpose(1, 0, 2).reshape(B, N, D)
    y = yout.reshape(B, E, D)
    return (x, y)


# speedup vs baseline: 1.0442x; 1.0442x over previous
import functools

import jax
import jax.numpy as jnp
from jax import lax
from jax.experimental import pallas as pl
from jax.experimental.pallas import tpu as pltpu
from jax.experimental.pallas import tpu_sc as plsc

F32 = jnp.float32
NC, NS = 2, 16
CE = 40
NODE_SUB = 25


def _node_proj_body(x, wsg, bsg, wdg, bdg, wdu, bdu, wsu, bsu,
                    osg, odg, odu, osu):
    xv = x[...]

    def proj(w, b):
        return lax.dot_general(xv, w[...], (((1,), (1,)), ((), ())),
                               preferred_element_type=F32) + b[...]

    osg[...] = proj(wsg, bsg)
    odg[...] = proj(wdg, bdg)
    odu[...] = proj(wdu, bdu)
    osu[...] = proj(wsu, bsu)


def _edge_proj_body(x, w, b, o):
    o[...] = lax.dot_general(x[...], w[...], (((1,), (1,)), ((), ())),
                             preferred_element_type=F32) + b[...]


def _sc_body(N, E, ti0, ti1, tj0, tj1, psut, node_t, peg, edge, ij_hbm,
             yout, xout,
             acc2,
             ijb0, ijb1, ijb2, ijb3,
             g_sg0, g_sg1, g_dj0, g_dj1,
             sm0, sm1, pg0, pg1, yo0, yo1,
             ep_sm, ep_psu, ep_node,
             sem_ij, sem_in0, sem_in1, sem_out0, sem_out1, sem_sc0, sem_sc1):
    c = lax.axis_index("c")
    s = lax.axis_index("s")
    cN = c * N
    c64 = c * 64
    zero16 = jnp.zeros((16,), F32)

    ijb = (ijb0, ijb1, ijb2, ijb3)
    g_sg = (g_sg0, g_sg1)
    g_dj = (g_dj0, g_dj1)
    sm = (sm0, sm1)
    pg = (pg0, pg1)
    yo = (yo0, yo1)
    sem_in = (sem_in0, sem_in1)
    sem_out = (sem_out0, sem_out1)
    sem_sc = (sem_sc0, sem_sc1)

    npt = N // NS
    ept = E // NS
    nchunks = ept // CE
    nsub = npt // NODE_SUB
    ebase0 = s * ept

    def zrow(r, x_):
        for q in range(8):
            ep_sm[r, pl.ds(q * 16, 16)] = zero16
        return x_
    lax.fori_loop(0, NODE_SUB, zrow, None)
    for q in range(nsub):
        pltpu.sync_copy(ep_sm, acc2.at[pl.ds(s * npt + q * NODE_SUB, NODE_SUB)])
    plsc.subcore_barrier()

    nct = ept // CE
    kbase0 = s * nct

    def issue_ij(k, q4):
        pltpu.async_copy(ij_hbm.at[kbase0 + k], ijb[q4], sem_ij)

    def wait_ij(q4):
        pltpu.make_async_copy(ij_hbm.at[kbase0], ijb[q4], sem_ij).wait()

    def issue_in(k, p, q4):
        base = ebase0 + k * CE

        @pl.when(c == 0)
        def _():
            pltpu.async_copy(ti0.at[ijb[q4].at[0]], g_sg[p], sem_in[p])
            pltpu.async_copy(tj0.at[ijb[q4].at[1]], g_dj[p], sem_in[p])

        @pl.when(c == 1)
        def _():
            pltpu.async_copy(ti1.at[ijb[q4].at[0]], g_sg[p], sem_in[p])
            pltpu.async_copy(tj1.at[ijb[q4].at[1]], g_dj[p], sem_in[p])

        pltpu.async_copy(peg.at[pl.ds(base, CE), pl.ds(c64, 64)], pg[p], sem_in[p])
        pltpu.async_copy(edge.at[pl.ds(base, CE), pl.ds(c64, 64)], yo[p], sem_in[p])

    def wait_in(p):
        pltpu.make_async_copy(ti0.at[ijb[0].at[0]], g_sg[p], sem_in[p]).wait()
        pltpu.make_async_copy(tj0.at[ijb[0].at[1]], g_dj[p], sem_in[p]).wait()
        pltpu.make_async_copy(peg.at[pl.ds(0, CE), pl.ds(c64, 64)], pg[p], sem_in[p]).wait()
        pltpu.make_async_copy(edge.at[pl.ds(0, CE), pl.ds(c64, 64)], yo[p], sem_in[p]).wait()

    def compute(p):
        def row2(t, x_):
            vals = []
            for rr in range(2):
                r = 2 * t + rr
                ys = [g_sg[p][r, pl.ds(q * 16, 16)]
                      + g_dj[p][r, pl.ds(q * 16, 16)]
                      + pg[p][r, pl.ds(q * 16, 16)] for q in range(4)]
                es = [yo[p][r, pl.ds(q * 16, 16)] for q in range(4)]
                dus = [g_dj[p][r, pl.ds(64 + q * 16, 16)] for q in range(4)]
                vals.append((r, ys, es, dus))
            for r, ys, es, dus in vals:
                for q in range(4):
                    sg = 1.0 / (1.0 + jnp.exp(-ys[q]))
                    sm[p][r, pl.ds(q * 16, 16)] = sg
                    sm[p][r, pl.ds(64 + q * 16, 16)] = dus[q] * sg
                    yo[p][r, pl.ds(q * 16, 16)] = es[q] + ys[q] * sg
            return x_
        lax.fori_loop(0, CE // 2, row2, None)

    def issue_out(k, p, q4):
        base = ebase0 + k * CE
        pltpu.async_copy(yo[p], yout.at[pl.ds(base, CE), pl.ds(c64, 64)], sem_out[p])
        pltpu.async_copy(sm[p], acc2.at[ijb[q4].at[0]], sem_sc[p], add=True)

    def wait_out(p):
        pltpu.make_async_copy(yo[p], yout.at[pl.ds(0, CE), pl.ds(c64, 64)], sem_out[p]).wait()

    def wait_sc(p):
        pltpu.make_async_copy(sm[p], acc2.at[ijb[0].at[0]], sem_sc[p]).wait()

    issue_ij(0, 0)
    issue_ij(1, 1)
    wait_ij(0)
    issue_in(0, 0, 0)

    wait_ij(1)
    issue_in(1, 1, 1)
    wait_in(0)
    compute(0)
    issue_out(0, 0, 0)
    issue_ij(2, 2)

    wait_ij(2)
    wait_out(0)
    issue_in(2, 0, 2)
    wait_in(1)
    compute(1)
    issue_out(1, 1, 1)
    issue_ij(3, 3)

    def superstep(t, x_):
        k0 = 2 + t * 4

        def stage(koff, p, q4):
            k = k0 + koff
            wait_ij((q4 + 1) % 4)
            wait_out(1 - p)
            issue_in(k + 1, 1 - p, (q4 + 1) % 4)
            wait_in(p)
            wait_sc(p)
            compute(p)
            issue_out(k, p, q4)
            issue_ij(k + 2, (q4 + 2) % 4)

        stage(0, 0, 2)
        stage(1, 1, 3)
        stage(2, 0, 0)
        stage(3, 1, 1)
        return x_

    lax.fori_loop(0, (nchunks - 4) // 4, superstep, None)

    wait_ij(3)
    wait_out(1)
    issue_in(nchunks - 1, 1, 3)
    wait_in(0)
    wait_sc(0)
    compute(0)
    issue_out(nchunks - 2, 0, 2)

    wait_in(1)
    wait_sc(1)
    compute(1)
    issue_out(nchunks - 1, 1, 3)

    wait_out(0)
    wait_out(1)
    wait_sc(0)
    wait_sc(1)
    plsc.subcore_barrier()

    nbase0 = s * npt

    def ep(q, x_):
        nb = nbase0 + q * NODE_SUB
        pltpu.sync_copy(acc2.at[pl.ds(nb, NODE_SUB)], ep_sm)
        pltpu.sync_copy(psut.at[pl.ds(cN + nb, NODE_SUB)], ep_psu)
        pltpu.sync_copy(node_t.at[pl.ds(cN + nb, NODE_SUB)], ep_node)

        def nrow(r, y_):
            sig4 = [ep_sm[r, pl.ds(q4 * 16, 16)] for q4 in range(4)]
            m4 = [ep_sm[r, pl.ds(64 + q4 * 16, 16)] for q4 in range(4)]
            psu4 = [ep_psu[r, pl.ds(q4 * 16, 16)] for q4 in range(4)]
            nd4 = [ep_node[r, pl.ds(q4 * 16, 16)] for q4 in range(4)]
            for q4 in range(4):
                t = psu4[q4] + m4[q4] / (sig4[q4] + 1e-6)
                st = 1.0 / (1.0 + jnp.exp(-t))
                ep_node[r, pl.ds(q4 * 16, 16)] = nd4[q4] + t * st
            return y_
        lax.fori_loop(0, NODE_SUB, nrow, None)

        pltpu.sync_copy(ep_node, xout.at[pl.ds(cN + nb, NODE_SUB)])
        return x_

    lax.fori_loop(0, nsub, ep, None)


def kernel(node_feats, edge_feats, i, j, W_sg, b_sg, W_dg, b_dg,
           W_eg, b_eg, W_su, b_su, W_du, b_du):
    B, N, D = node_feats.shape
    E = edge_feats.shape[1]
    node2 = node_feats.reshape(N, D)
    edge2 = edge_feats.reshape(E, D)
    ij_hbm = jnp.stack([i.astype(jnp.int32).reshape(E // CE, CE),
                        j.astype(jnp.int32).reshape(E // CE, CE)], axis=1)
    h = D // 2

    BN = 400
    wspec = pl.BlockSpec((D, D), lambda nb: (0, 0))
    bspec = pl.BlockSpec((1, D), lambda nb: (0, 0))
    outs = pl.pallas_call(
        _node_proj_body,
        grid=(N // BN,),
        in_specs=[pl.BlockSpec((BN, D), lambda nb: (nb, 0)),
                  wspec, bspec, wspec, bspec, wspec, bspec, wspec, bspec],
        out_specs=[pl.BlockSpec((BN, D), lambda nb: (nb, 0))] * 4,
        out_shape=[jax.ShapeDtypeStruct((N, D), F32)] * 4,
    )(node2, W_sg, b_sg.reshape(1, D), W_dg, b_dg.reshape(1, D),
      W_du, b_du.reshape(1, D), W_su, b_su.reshape(1, D))
    p_sg, p_dg, p_du, p_su = outs

    ti0 = p_sg[:, :h]
    ti1 = p_sg[:, h:]
    tj0 = jnp.concatenate([p_dg[:, :h], p_du[:, :h]], axis=1)
    tj1 = jnp.concatenate([p_dg[:, h:], p_du[:, h:]], axis=1)
    psut = p_su.reshape(N, 2, h).transpose(1, 0, 2).reshape(2 * N, h)
    node_t = node2.reshape(N, 2, h).transpose(1, 0, 2).reshape(2 * N, h)

    BE = 1000
    peg = pl.pallas_call(
        _edge_proj_body,
        grid=(E // BE,),
        in_specs=[pl.BlockSpec((BE, D), lambda nb: (nb, 0)),
                  pl.BlockSpec((D, D), lambda nb: (0, 0)),
                  pl.BlockSpec((1, D), lambda nb: (0, 0))],
        out_specs=pl.BlockSpec((BE, D), lambda nb: (nb, 0)),
        out_shape=jax.ShapeDtypeStruct((E, D), F32),
    )(edge2, W_eg, b_eg.reshape(1, D))

    mesh = plsc.VectorSubcoreMesh(core_axis_name="c", subcore_axis_name="s",
                                  num_cores=NC, num_subcores=NS)
    sc_fn = pl.kernel(
        functools.partial(_sc_body, N, E),
        out_type=[jax.ShapeDtypeStruct((E, D), F32),
                  jax.ShapeDtypeStruct((2 * N, h), F32)],
        mesh=mesh,
        compiler_params=pltpu.CompilerParams(use_tc_tiling_on_sc=False),
        scratch_types=[
            pltpu.VMEM_SHARED((N, D), F32),
            pltpu.VMEM((2, CE), jnp.int32),
            pltpu.VMEM((2, CE), jnp.int32),
            pltpu.VMEM((2, CE), jnp.int32),
            pltpu.VMEM((2, CE), jnp.int32),
            pltpu.VMEM((CE, h), F32),
            pltpu.VMEM((CE, h), F32),
            pltpu.VMEM((CE, D), F32),
            pltpu.VMEM((CE, D), F32),
            pltpu.VMEM((CE, D), F32),
            pltpu.VMEM((CE, D), F32),
            pltpu.VMEM((CE, h), F32),
            pltpu.VMEM((CE, h), F32),
            pltpu.VMEM((CE, h), F32),
            pltpu.VMEM((CE, h), F32),
            pltpu.VMEM((NODE_SUB, D), F32),
            pltpu.VMEM((NODE_SUB, h), F32),
            pltpu.VMEM((NODE_SUB, h), F32),
            pltpu.SemaphoreType.DMA,
            pltpu.SemaphoreType.DMA,
            pltpu.SemaphoreType.DMA,
            pltpu.SemaphoreType.DMA,
            pltpu.SemaphoreType.DMA,
            pltpu.SemaphoreType.DMA,
            pltpu.SemaphoreType.DMA,
        ],
    )
    yout, xout = sc_fn(ti0, ti1, tj0, tj1, psut, node_t, peg, edge2, ij_hbm)

    x = xout.reshape(2, N, h).transpose(1, 0, 2).reshape(B, N, D)
    y = yout.reshape(B, E, D)
    return (x, y)


# speedup vs baseline: 1.0799x; 1.0342x over previous
import functools

import jax
import jax.numpy as jnp
from jax import lax
from jax.experimental import pallas as pl
from jax.experimental.pallas import tpu as pltpu
from jax.experimental.pallas import tpu_sc as plsc

F32 = jnp.float32
NC, NS = 2, 16
CE = 40
NODE_SUB = 25


def _node_proj_body(x, wsg, bsg, wdg, bdg, wdu, bdu, wsu, bsu,
                    osg, odg, odu, osu):
    xv = x[...]

    def proj(w, b):
        return lax.dot_general(xv, w[...], (((1,), (1,)), ((), ())),
                               preferred_element_type=F32) + b[...]

    osg[...] = proj(wsg, bsg)
    odg[...] = proj(wdg, bdg)
    odu[...] = proj(wdu, bdu)
    osu[...] = proj(wsu, bsu)


def _edge_proj_body(x, w, b, o):
    o[...] = lax.dot_general(x[...], w[...], (((1,), (1,)), ((), ())),
                             preferred_element_type=F32) + b[...]


def _sc_body(N, E, ti0, ti1, tj0, tj1, psut, node_t, peg, edge, i_hbm, j_hbm,
             yout, xout,
             acc2,
             ib0, ib1, ib2, ib3, jb0, jb1, jb2, jb3,
             g_sg0, g_sg1, g_dj0, g_dj1,
             sm0, sm1, pg0, pg1, yo0, yo1,
             ep_sm, ep_psu, ep_node,
             sem_ij, sem_in0, sem_in1, sem_out0, sem_out1, sem_sc0, sem_sc1):
    c = lax.axis_index("c")
    s = lax.axis_index("s")
    cN = c * N
    c64 = c * 64
    zero16 = jnp.zeros((16,), F32)

    ib = (ib0, ib1, ib2, ib3)
    jb = (jb0, jb1, jb2, jb3)
    g_sg = (g_sg0, g_sg1)
    g_dj = (g_dj0, g_dj1)
    sm = (sm0, sm1)
    pg = (pg0, pg1)
    yo = (yo0, yo1)
    sem_in = (sem_in0, sem_in1)
    sem_out = (sem_out0, sem_out1)
    sem_sc = (sem_sc0, sem_sc1)

    npt = N // NS
    ept = E // NS
    nchunks = ept // CE
    nsub = npt // NODE_SUB
    ebase0 = s * ept

    def zrow(r, x_):
        for q in range(8):
            ep_sm[r, pl.ds(q * 16, 16)] = zero16
        return x_
    lax.fori_loop(0, NODE_SUB, zrow, None)
    for q in range(nsub):
        pltpu.sync_copy(ep_sm, acc2.at[pl.ds(s * npt + q * NODE_SUB, NODE_SUB)])
    plsc.subcore_barrier()

    def issue_ij(k, q4):
        base = ebase0 + k * CE
        pltpu.async_copy(i_hbm.at[pl.ds(base, CE)], ib[q4], sem_ij)
        pltpu.async_copy(j_hbm.at[pl.ds(base, CE)], jb[q4], sem_ij)

    def wait_ij(q4):
        pltpu.make_async_copy(i_hbm.at[pl.ds(0, CE)], ib[q4], sem_ij).wait()
        pltpu.make_async_copy(j_hbm.at[pl.ds(0, CE)], jb[q4], sem_ij).wait()

    def issue_in(k, p, q4):
        base = ebase0 + k * CE

        @pl.when(c == 0)
        def _():
            pltpu.async_copy(ti0.at[ib[q4]], g_sg[p], sem_in[p])
            pltpu.async_copy(tj0.at[jb[q4]], g_dj[p], sem_in[p])

        @pl.when(c == 1)
        def _():
            pltpu.async_copy(ti1.at[ib[q4]], g_sg[p], sem_in[p])
            pltpu.async_copy(tj1.at[jb[q4]], g_dj[p], sem_in[p])

        pltpu.async_copy(peg.at[pl.ds(base, CE), pl.ds(c64, 64)], pg[p], sem_in[p])
        pltpu.async_copy(edge.at[pl.ds(base, CE), pl.ds(c64, 64)], yo[p], sem_in[p])

    def wait_in(p):
        pltpu.make_async_copy(ti0.at[ib[0]], g_sg[p], sem_in[p]).wait()
        pltpu.make_async_copy(tj0.at[jb[0]], g_dj[p], sem_in[p]).wait()
        pltpu.make_async_copy(peg.at[pl.ds(0, CE), pl.ds(c64, 64)], pg[p], sem_in[p]).wait()
        pltpu.make_async_copy(edge.at[pl.ds(0, CE), pl.ds(c64, 64)], yo[p], sem_in[p]).wait()

    def compute(p):
        def row2(t, x_):
            vals = []
            for rr in range(2):
                r = 2 * t + rr
                ys = [g_sg[p][r, pl.ds(q * 16, 16)]
                      + g_dj[p][r, pl.ds(q * 16, 16)]
                      + pg[p][r, pl.ds(q * 16, 16)] for q in range(4)]
                es = [yo[p][r, pl.ds(q * 16, 16)] for q in range(4)]
                dus = [g_dj[p][r, pl.ds(64 + q * 16, 16)] for q in range(4)]
                vals.append((r, ys, es, dus))
            for r, ys, es, dus in vals:
                for q in range(4):
                    sg = 1.0 / (1.0 + jnp.exp(-ys[q]))
                    sm[p][r, pl.ds(q * 16, 16)] = sg
                    sm[p][r, pl.ds(64 + q * 16, 16)] = dus[q] * sg
                    yo[p][r, pl.ds(q * 16, 16)] = es[q] + ys[q] * sg
            return x_
        lax.fori_loop(0, CE // 2, row2, None)

    def issue_out(k, p, q4):
        base = ebase0 + k * CE
        pltpu.async_copy(yo[p], yout.at[pl.ds(base, CE), pl.ds(c64, 64)], sem_out[p])
        pltpu.async_copy(sm[p], acc2.at[ib[q4]], sem_sc[p], add=True)

    def wait_out(p):
        pltpu.make_async_copy(yo[p], yout.at[pl.ds(0, CE), pl.ds(c64, 64)], sem_out[p]).wait()

    def wait_sc(p):
        pltpu.make_async_copy(sm[p], acc2.at[ib[0]], sem_sc[p]).wait()

    issue_ij(0, 0)
    issue_ij(1, 1)
    wait_ij(0)
    issue_in(0, 0, 0)

    wait_ij(1)
    issue_in(1, 1, 1)
    wait_in(0)
    compute(0)
    issue_out(0, 0, 0)
    issue_ij(2, 2)

    wait_ij(2)
    wait_out(0)
    issue_in(2, 0, 2)
    wait_in(1)
    compute(1)
    issue_out(1, 1, 1)
    issue_ij(3, 3)

    def superstep(t, x_):
        k0 = 2 + t * 4

        def stage(koff, p, q4):
            k = k0 + koff
            wait_ij((q4 + 1) % 4)
            wait_out(1 - p)
            issue_in(k + 1, 1 - p, (q4 + 1) % 4)
            wait_in(p)
            wait_sc(p)
            compute(p)
            issue_out(k, p, q4)
            issue_ij(k + 2, (q4 + 2) % 4)

        stage(0, 0, 2)
        stage(1, 1, 3)
        stage(2, 0, 0)
        stage(3, 1, 1)
        return x_

    lax.fori_loop(0, (nchunks - 4) // 4, superstep, None)

    wait_ij(3)
    wait_out(1)
    issue_in(nchunks - 1, 1, 3)
    wait_in(0)
    wait_sc(0)
    compute(0)
    issue_out(nchunks - 2, 0, 2)

    wait_in(1)
    wait_sc(1)
    compute(1)
    issue_out(nchunks - 1, 1, 3)

    wait_out(0)
    wait_out(1)
    wait_sc(0)
    wait_sc(1)
    plsc.subcore_barrier()

    nbase0 = s * npt

    def ep(q, x_):
        nb = nbase0 + q * NODE_SUB
        pltpu.sync_copy(acc2.at[pl.ds(nb, NODE_SUB)], ep_sm)
        pltpu.sync_copy(psut.at[pl.ds(cN + nb, NODE_SUB)], ep_psu)
        pltpu.sync_copy(node_t.at[pl.ds(cN + nb, NODE_SUB)], ep_node)

        def nrow(r, y_):
            sig4 = [ep_sm[r, pl.ds(q4 * 16, 16)] for q4 in range(4)]
            m4 = [ep_sm[r, pl.ds(64 + q4 * 16, 16)] for q4 in range(4)]
            psu4 = [ep_psu[r, pl.ds(q4 * 16, 16)] for q4 in range(4)]
            nd4 = [ep_node[r, pl.ds(q4 * 16, 16)] for q4 in range(4)]
            for q4 in range(4):
                t = psu4[q4] + m4[q4] / (sig4[q4] + 1e-6)
                st = 1.0 / (1.0 + jnp.exp(-t))
                ep_node[r, pl.ds(q4 * 16, 16)] = nd4[q4] + t * st
            return y_
        lax.fori_loop(0, NODE_SUB, nrow, None)

        pltpu.sync_copy(ep_node, xout.at[pl.ds(cN + nb, NODE_SUB)])
        return x_

    lax.fori_loop(0, nsub, ep, None)


def kernel(node_feats, edge_feats, i, j, W_sg, b_sg, W_dg, b_dg,
           W_eg, b_eg, W_su, b_su, W_du, b_du):
    B, N, D = node_feats.shape
    E = edge_feats.shape[1]
    node2 = node_feats.reshape(N, D)
    edge2 = edge_feats.reshape(E, D)
    i32 = i.astype(jnp.int32)
    j32 = j.astype(jnp.int32)
    h = D // 2

    BN = 400
    wspec = pl.BlockSpec((D, D), lambda nb: (0, 0))
    bspec = pl.BlockSpec((1, D), lambda nb: (0, 0))
    outs = pl.pallas_call(
        _node_proj_body,
        grid=(N // BN,),
        in_specs=[pl.BlockSpec((BN, D), lambda nb: (nb, 0)),
                  wspec, bspec, wspec, bspec, wspec, bspec, wspec, bspec],
        out_specs=[pl.BlockSpec((BN, D), lambda nb: (nb, 0))] * 4,
        out_shape=[jax.ShapeDtypeStruct((N, D), F32)] * 4,
    )(node2, W_sg, b_sg.reshape(1, D), W_dg, b_dg.reshape(1, D),
      W_du, b_du.reshape(1, D), W_su, b_su.reshape(1, D))
    p_sg, p_dg, p_du, p_su = outs

    ti0 = p_sg[:, :h]
    ti1 = p_sg[:, h:]
    tj0 = jnp.concatenate([p_dg[:, :h], p_du[:, :h]], axis=1)
    tj1 = jnp.concatenate([p_dg[:, h:], p_du[:, h:]], axis=1)
    psut = p_su.reshape(N, 2, h).transpose(1, 0, 2).reshape(2 * N, h)
    node_t = node2.reshape(N, 2, h).transpose(1, 0, 2).reshape(2 * N, h)

    BE = 1000
    peg = pl.pallas_call(
        _edge_proj_body,
        grid=(E // BE,),
        in_specs=[pl.BlockSpec((BE, D), lambda nb: (nb, 0)),
                  pl.BlockSpec((D, D), lambda nb: (0, 0)),
                  pl.BlockSpec((1, D), lambda nb: (0, 0))],
        out_specs=pl.BlockSpec((BE, D), lambda nb: (nb, 0)),
        out_shape=jax.ShapeDtypeStruct((E, D), F32),
    )(edge2, W_eg, b_eg.reshape(1, D))

    mesh = plsc.VectorSubcoreMesh(core_axis_name="c", subcore_axis_name="s",
                                  num_cores=NC, num_subcores=NS)
    sc_fn = pl.kernel(
        functools.partial(_sc_body, N, E),
        out_type=[jax.ShapeDtypeStruct((E, D), F32),
                  jax.ShapeDtypeStruct((2 * N, h), F32)],
        mesh=mesh,
        compiler_params=pltpu.CompilerParams(use_tc_tiling_on_sc=False),
        scratch_types=[
            pltpu.VMEM_SHARED((N, D), F32),
            pltpu.VMEM((CE,), jnp.int32),
            pltpu.VMEM((CE,), jnp.int32),
            pltpu.VMEM((CE,), jnp.int32),
            pltpu.VMEM((CE,), jnp.int32),
            pltpu.VMEM((CE,), jnp.int32),
            pltpu.VMEM((CE,), jnp.int32),
            pltpu.VMEM((CE,), jnp.int32),
            pltpu.VMEM((CE,), jnp.int32),
            pltpu.VMEM((CE, h), F32),
            pltpu.VMEM((CE, h), F32),
            pltpu.VMEM((CE, D), F32),
            pltpu.VMEM((CE, D), F32),
            pltpu.VMEM((CE, D), F32),
            pltpu.VMEM((CE, D), F32),
            pltpu.VMEM((CE, h), F32),
            pltpu.VMEM((CE, h), F32),
            pltpu.VMEM((CE, h), F32),
            pltpu.VMEM((CE, h), F32),
            pltpu.VMEM((NODE_SUB, D), F32),
            pltpu.VMEM((NODE_SUB, h), F32),
            pltpu.VMEM((NODE_SUB, h), F32),
            pltpu.SemaphoreType.DMA,
            pltpu.SemaphoreType.DMA,
            pltpu.SemaphoreType.DMA,
            pltpu.SemaphoreType.DMA,
            pltpu.SemaphoreType.DMA,
            pltpu.SemaphoreType.DMA,
            pltpu.SemaphoreType.DMA,
        ],
    )
    yout, xout = sc_fn(ti0, ti1, tj0, tj1, psut, node_t, peg, edge2, i32, j32)

    x = xout.reshape(2, N, h).transpose(1, 0, 2).reshape(B, N, D)
    y = yout.reshape(B, E, D)
    return (x, y)
